# baseline (speedup 1.0000x reference)
import jax
import jax.numpy as jnp
from jax import lax
from jax.experimental import pallas as pl
from jax.experimental.pallas import tpu as pltpu
from jax.experimental.pallas import tpu_sc as plsc

_DELTA_V = 1.0
_DELTA_D = 6.0

_B = 8
_C = 8
_N = 144 * 256
_NSUB = 32
_P = (_B * _N) // _NSUB
_NV = _P // 16


def _vec_sqrt(s):
  ibits = plsc.bitcast(s, jnp.int32)
  y = plsc.bitcast(jnp.int32(0x5F3759DF) - (ibits >> 1), jnp.float32)
  for _ in range(3):
    y = y * (1.5 - 0.5 * s * y * y)
  return s * y


def _sc_body(e_hbm, t_hbm, out_hbm, e_v, t_v, acc_v, comb_v, pub_s, mean_v,
             w_v, outv_v):
  c_id = lax.axis_index("c")
  s_id = lax.axis_index("s")
  wid = c_id * 16 + s_id
  b = c_id * 4 + s_id // 4
  q = s_id % 4
  base = q * _P

  pltpu.sync_copy(e_hbm.at[b, :, pl.ds(base, _P)], e_v)
  pltpu.sync_copy(t_hbm.at[b, pl.ds(base, _P)], t_v)

  zeros = jnp.zeros((16,), jnp.float32)

  def slot(l, j):
    return pl.ds((l * 9 + j) * 16, 16)

  for l in range(6):
    for j in range(9):
      acc_v[slot(l, j)] = zeros

  def p1(i, carry):
    off = pl.multiple_of(i * 16, 16)
    t_vec = t_v[pl.ds(off, 16)]
    for l in range(1, 6):
      m = t_vec == l
      for c in range(_C):
        e_vec = e_v[c, pl.ds(off, 16)]
        plsc.addupdate(acc_v.at[slot(l, c)], jnp.where(m, e_vec, 0.0))
      plsc.addupdate(acc_v.at[slot(l, 8)], jnp.where(m, 1.0, 0.0))
    return carry

  lax.fori_loop(0, _NV, p1, jnp.float32(0.0))

  pltpu.sync_copy(acc_v, pub_s.at[s_id])
  plsc.subcore_barrier()
  g0 = (s_id // 4) * 4
  pltpu.sync_copy(pub_s.at[g0], comb_v)
  for k in range(1, 4):
    pltpu.sync_copy(pub_s.at[g0 + k], acc_v)
    for l in range(1, 6):
      for j in range(9):
        comb_v[slot(l, j)] = comb_v[slot(l, j)] + acc_v[slot(l, j)]

  p = lax.iota(jnp.int32, 16)

  def splat(x):
    return jnp.broadcast_to(x, (16,))

  mean_v[pl.ds(0, 16)] = zeros
  w_row = zeros
  pc_vec = zeros
  for l in range(1, 6):
    cntv = splat(jnp.sum(comb_v[slot(l, 8)]))
    safe = jnp.maximum(cntv, 1.0)
    w_lv = jnp.where(cntv > 1.5, 1.0, 0.0).astype(jnp.float32)
    w_row = jnp.where(p == l, w_lv, w_row)
    pc_vec = pc_vec + w_lv * cntv
    row = zeros
    for c in range(_C):
      row = jnp.where(p == c, splat(jnp.sum(comb_v[slot(l, c)])), row)
    mean_v[pl.ds(l * 16, 16)] = row / safe
  w_v[...] = w_row

  def p2(i, acc):
    off = pl.multiple_of(i * 16, 16)
    t_vec = t_v[pl.ds(off, 16)]
    idx16 = t_vec * 16
    s = jnp.full((16,), 1e-12, jnp.float32)
    for c in range(_C):
      e_vec = e_v[c, pl.ds(off, 16)]
      m_vec = plsc.load_gather(mean_v, [idx16 + c])
      d = e_vec - m_vec
      s = s + d * d
    n = _vec_sqrt(s)
    h = jnp.maximum(n - _DELTA_V, 0.0)
    wvec = plsc.load_gather(w_v, [t_vec])
    return acc + wvec * h * h

  dist_acc = lax.fori_loop(0, _NV, p2, zeros)
  tot_dist_part = jnp.sum(dist_acc)

  ge4 = jnp.where(p >= 4, 1, 0)
  ge7 = jnp.where(p >= 7, 1, 0)
  ge9 = jnp.where(p >= 9, 1, 0)
  i_vec = ge4 + ge7 + ge9 + 1
  j_vec = jnp.minimum(p + 2 - (3 * ge4 + 2 * ge7 + ge9), 5)
  d2 = jnp.full((16,), 1e-12, jnp.float32)
  for c in range(_C):
    va = plsc.load_gather(mean_v, [i_vec * 16 + c])
    vb = plsc.load_gather(mean_v, [j_vec * 16 + c])
    dd = va - vb
    d2 = d2 + dd * dd
  pd = _vec_sqrt(d2)
  pen = jnp.maximum(_DELTA_D - pd, 0.0)
  pen = pen * pen
  wi = plsc.load_gather(w_v, [i_vec])
  wj = plsc.load_gather(w_v, [j_vec])
  pm = wi * wj * jnp.where(p < 10, 1.0, 0.0)
  pen_sum_v = splat(jnp.sum(pen * pm))
  n_pairs_v = splat(jnp.sum(pm))
  tv_v = jnp.where(n_pairs_v > 0.5,
                   pen_sum_v / jnp.maximum(n_pairs_v, 1.0), 0.0)

  flag_v = jnp.where(splat(q) == 0, 1.0, 0.0).astype(jnp.float32)
  outvec = jnp.where(p == 0, splat(tot_dist_part),
                     jnp.where(p == 1, flag_v * tv_v,
                               jnp.where(p == 2, flag_v * pc_vec, 0.0)))
  outv_v[...] = outvec
  pltpu.sync_copy(outv_v, out_hbm.at[wid])


@jax.jit
def kernel(targets, embedding_vector):
  e = embedding_vector.reshape(_B, _C, _N)
  t = targets.reshape(_B, _N)

  mesh = plsc.VectorSubcoreMesh(core_axis_name="c", subcore_axis_name="s",
                                num_cores=2, num_subcores=16)
  parts = pl.kernel(
      _sc_body,
      out_type=jax.ShapeDtypeStruct((_NSUB, 16), jnp.float32),
      mesh=mesh,
      compiler_params=pltpu.CompilerParams(needs_layout_passes=False),
      scratch_types=[
          pltpu.VMEM((_C, _P), jnp.float32),
          pltpu.VMEM((_P,), jnp.int32),
          pltpu.VMEM((864,), jnp.float32),
          pltpu.VMEM((864,), jnp.float32),
          pltpu.VMEM_SHARED((16, 864), jnp.float32),
          pltpu.VMEM((96,), jnp.float32),
          pltpu.VMEM((16,), jnp.float32),
          pltpu.VMEM((16,), jnp.float32),
      ],
  )(e, t)

  tot_dist = jnp.sum(parts[:, 0])
  tot_var = jnp.sum(parts[:, 1])
  pc = jnp.sum(parts[:, 2])
  dist_term = jnp.where(pc > 0, tot_dist / jnp.maximum(pc, 1.0), 0.0)
  loss = dist_term + tot_var / _B
  return loss.reshape(1)
